# Initial kernel scaffold; baseline (speedup 1.0000x reference)
import jax
import jax.numpy as jnp
from jax.experimental import pallas as pl


def kernel(x, emb_table, W, b):
    raise NotImplementedError("write your pallas kernel here")


# speedup vs baseline: 1.1773x; 1.1773x over previous
import functools

import jax
import jax.numpy as jnp
from jax import lax
from jax.experimental import pallas as pl
from jax.experimental.pallas import tpu as pltpu
from jax.experimental.pallas import tpu_sc as plsc

_B = 4096
_EMB = 64
_LANES = 16

_info = plsc.get_sparse_core_info()
_NC = _info.num_cores
_NS = _info.num_subcores
_NW = _NC * _NS
_BPW = _B // _NW

_mesh = plsc.VectorSubcoreMesh(core_axis_name="c", subcore_axis_name="s")


@functools.partial(
    pl.kernel,
    mesh=_mesh,
    out_type=jax.ShapeDtypeStruct((_B,), jnp.float32),
    compiler_params=pltpu.CompilerParams(
        needs_layout_passes=False, use_tc_tiling_on_sc=False),
    scratch_types=[
        pltpu.VMEM((_BPW,), jnp.int32),
        pltpu.VMEM((_BPW, _EMB), jnp.float32),
        pltpu.VMEM((_EMB + _LANES,), jnp.float32),
        pltpu.VMEM((_BPW,), jnp.float32),
        pltpu.SemaphoreType.DMA,
    ],
)
def _sc_forward(table_hbm, idx_hbm, wb_hbm, out_hbm,
                idx_v, rows_v, wb_v, res_v, sem):
    wid = lax.axis_index("s") * _NC + lax.axis_index("c")
    base = wid * _BPW

    pltpu.sync_copy(idx_hbm.at[pl.ds(base, _BPW)], idx_v)
    pltpu.sync_copy(wb_hbm, wb_v)
    pltpu.async_copy(table_hbm.at[idx_v], rows_v, sem).wait()

    lane = lax.iota(jnp.int32, _LANES)
    bias = wb_v[pl.ds(_EMB, _LANES)]
    w_vecs = [wb_v[pl.ds(c * _LANES, _LANES)] for c in range(_EMB // _LANES)]
    w_s = [w_vecs[d // _LANES][d % _LANES] for d in range(_EMB)]

    for g in range(_BPW // _LANES):
        row_ids = lane + (g * _LANES)
        acc = bias
        for d in range(_EMB):
            col = plsc.load_gather(
                rows_v, [row_ids, jnp.full((_LANES,), d, jnp.int32)])
            acc = acc + col * w_s[d]
        res_v[pl.ds(g * _LANES, _LANES)] = 1.0 / (1.0 + jnp.exp(-acc))

    pltpu.sync_copy(res_v, out_hbm.at[pl.ds(base, _BPW)])


def kernel(x, emb_table, W, b):
    idx = x[:, -1].astype(jnp.int32)
    wb = jnp.concatenate([
        W.reshape(-1).astype(jnp.float32),
        jnp.broadcast_to(b.astype(jnp.float32).reshape(-1)[:1], (_LANES,)),
    ])
    return _sc_forward(emb_table, idx, wb)


# speedup vs baseline: 10.1672x; 8.6357x over previous
import functools

import jax
import jax.numpy as jnp
from jax import lax
from jax.experimental import pallas as pl
from jax.experimental.pallas import tpu as pltpu
from jax.experimental.pallas import tpu_sc as plsc

_B = 4096
_EMB = 64
_LANES = 16
_VOCAB_TILE = 128

_info = plsc.get_sparse_core_info()
_NC = _info.num_cores
_NS = _info.num_subcores
_NW = _NC * _NS
_BPW = _B // _NW

_NCH = 4
_NCHUNKS = _BPW // _NCH
_NBUF = 2

_mesh = plsc.VectorSubcoreMesh(core_axis_name="c", subcore_axis_name="s")


@functools.partial(
    pl.kernel,
    mesh=_mesh,
    out_type=jax.ShapeDtypeStruct((_B,), jnp.float32),
    compiler_params=pltpu.CompilerParams(needs_layout_passes=False),
    scratch_types=[
        pltpu.VMEM((_BPW,), jnp.int32),
        pltpu.VMEM((_NBUF, _NCH, _EMB, _VOCAB_TILE), jnp.float32),
        pltpu.VMEM((_EMB + _LANES,), jnp.float32),
        pltpu.VMEM((_BPW * _LANES,), jnp.float32),
        pltpu.VMEM((_BPW,), jnp.float32),
        pltpu.SemaphoreType.DMA,
    ],
)
def _sc_forward(table_t_hbm, idx_hbm, wb_hbm, out_hbm,
                idx_v, tiles_v, wb_v, par_v, res_v, sem):
    wid = lax.axis_index("s") * _NC + lax.axis_index("c")
    base = wid * _BPW

    pltpu.sync_copy(idx_hbm.at[pl.ds(base, _BPW)], idx_v)
    pltpu.sync_copy(wb_hbm, wb_v)

    lane = lax.iota(jnp.int32, _LANES)
    bias = wb_v[pl.ds(_EMB, _LANES)]
    w_vecs = [wb_v[pl.ds(c * _LANES, _LANES)] for c in range(_EMB // _LANES)]
    zeros16 = jnp.zeros((_LANES,), jnp.int32)

    def chunk_scalars(ch):
        vec = idx_v[pl.ds((ch // _NCH) * _LANES, _LANES)]
        return [vec[(ch % _NCH) * _NCH + q] for q in range(_NCH)]

    def fire(ch, slot):
        hs = []
        for q, i in enumerate(chunk_scalars(ch)):
            cb = pl.multiple_of((i >> 7) << 7, _VOCAB_TILE)
            hs.append(pltpu.async_copy(
                table_t_hbm.at[:, pl.ds(cb, _VOCAB_TILE)],
                tiles_v.at[slot, q], sem))
        return hs

    def process(ch, slot):
        for q, i in enumerate(chunk_scalars(ch)):
            j = ch * _NCH + q
            cvec = zeros16 + (i & (_VOCAB_TILE - 1))
            acc = jnp.zeros((_LANES,), jnp.float32)
            for dg in range(_EMB // _LANES):
                v = plsc.load_gather(tiles_v.at[slot, q],
                                     [lane + dg * _LANES, cvec])
                acc = acc + v * w_vecs[dg]
            par_v[pl.ds(j * _LANES, _LANES)] = acc

    handles = fire(0, 0)
    for ch in range(_NCHUNKS):
        nxt = None
        if ch + 1 < _NCHUNKS:
            nxt = fire(ch + 1, (ch + 1) % _NBUF)
        for h in handles:
            h.wait()
        process(ch, ch % _NBUF)
        handles = nxt

    for g in range(_BPW // _LANES):
        acc = bias
        for l in range(_LANES):
            acc = acc + plsc.load_gather(
                par_v, [lane * _LANES + (g * _LANES * _LANES + l)])
        res_v[pl.ds(g * _LANES, _LANES)] = 1.0 / (1.0 + jnp.exp(-acc))

    pltpu.sync_copy(res_v, out_hbm.at[pl.ds(base, _BPW)])


def kernel(x, emb_table, W, b):
    idx = x[:, -1].astype(jnp.int32)
    wb = jnp.concatenate([
        W.reshape(-1).astype(jnp.float32),
        jnp.broadcast_to(b.astype(jnp.float32).reshape(-1)[:1], (_LANES,)),
    ])
    return _sc_forward(emb_table.T, idx, wb)
